# baseline (speedup 1.0000x reference)
import functools

import jax
import jax.numpy as jnp
from jax import lax
from jax.experimental import pallas as pl
from jax.experimental.pallas import tpu as pltpu
from jax.experimental.pallas import tpu_sc as plsc

_N = 10000
_NP = 10240
_E = 320000
_H = 128
_B = 64

_NC = 2
_NS = 16
_NW = _NC * _NS
_EPT = _E // _NW
_EK = 100
_ESTEPS = _EPT // _EK
_NBUF = 3
_NIB = 6
_RPT = _NP // _NS
_RK = 128
_RSTEPS = _RPT // _RK
_HPT = _NP // _NS

_RB = 1024
_GB = _NP // _RB

_mesh = plsc.VectorSubcoreMesh(core_axis_name="c", subcore_axis_name="s")


@functools.partial(
    pl.kernel,
    mesh=_mesh,
    out_type=jax.ShapeDtypeStruct((_NC, _NP), jnp.float32),
    scratch_types=[
        pltpu.VMEM((_HPT,), jnp.float32),
        pltpu.VMEM((112,), jnp.float32),
        pltpu.VMEM((_ESTEPS, _EK), jnp.int32),
        pltpu.SemaphoreType.DMA,
        pltpu.VMEM_SHARED((_NP,), jnp.float32),
    ],
)
def _deg_kernel(e3_hbm, out_hbm, zbuf, ones_v, dstall, dsem, hist):
    cid = lax.axis_index("c")
    sid = lax.axis_index("s")
    wid = cid * _NS + sid
    pltpu.sync_copy(e3_hbm.at[1, wid], dstall)
    for j in range(_HPT // 16):
        zbuf[pl.ds(j * 16, 16)] = jnp.zeros((16,), jnp.float32)
    for j in range(112 // 16):
        ones_v[pl.ds(j * 16, 16)] = jnp.ones((16,), jnp.float32)
    pltpu.sync_copy(zbuf, hist.at[pl.ds(sid * _HPT, _HPT)])
    plsc.subcore_barrier()
    ones_c = ones_v.at[pl.ds(0, _EK)]

    def fire(i, carry):
        pltpu.async_copy(ones_c, hist.at[dstall.at[i]], dsem, add=True)
        return carry

    def drain(i, carry):
        pltpu.make_async_copy(ones_c, hist.at[dstall.at[i]], dsem).wait()
        return carry

    lax.fori_loop(0, _ESTEPS, fire, 0)
    lax.fori_loop(0, _ESTEPS, drain, 0)
    plsc.subcore_barrier()
    pltpu.sync_copy(hist.at[pl.ds(sid * _HPT, _HPT)], zbuf)
    pltpu.sync_copy(zbuf, out_hbm.at[cid, pl.ds(sid * _HPT, _HPT)])


@functools.partial(
    pl.kernel,
    mesh=_mesh,
    out_type=jax.ShapeDtypeStruct((_NC, _NP, _H), jnp.float32),
    scratch_types=[
        pltpu.VMEM((_NBUF, _EK, _H), jnp.float32),
        pltpu.VMEM((_NIB, 2, _EK), jnp.int32),
    ] + [pltpu.SemaphoreType.DMA] * (2 * _NBUF + _NIB) + [
        pltpu.VMEM_SHARED((_NP, _H), jnp.float32),
    ],
)
def _agg_kernel(y_hbm, e3_hbm, out_hbm, erows, eidx,
                g0, g1, g2, s0, s1, s2,
                i0, i1, i2, i3, i4, i5, acc):
    gsem = [g0, g1, g2]
    ssem = [s0, s1, s2]
    isem = [i0, i1, i2, i3, i4, i5]
    cid = lax.axis_index("c")
    sid = lax.axis_index("s")
    wid = cid * _NS + sid
    r0 = sid * _RPT
    pltpu.sync_copy(y_hbm.at[pl.ds(r0, _RPT)], acc.at[pl.ds(r0, _RPT)])
    plsc.subcore_barrier()

    def i_start(j, m):
        pltpu.async_copy(e3_hbm.at[0, wid, j], eidx.at[m, 0], isem[m])
        pltpu.async_copy(e3_hbm.at[1, wid, j], eidx.at[m, 1], isem[m])

    def i_wait(j, m):
        pltpu.make_async_copy(e3_hbm.at[0, wid, j], eidx.at[m, 0],
                              isem[m]).wait()
        pltpu.make_async_copy(e3_hbm.at[1, wid, j], eidx.at[m, 1],
                              isem[m]).wait()

    _EH = _EK // 2
    def g_start(i, m, b):
        pltpu.async_copy(y_hbm.at[eidx.at[m, 0, pl.ds(0, _EH)]],
                         erows.at[b, pl.ds(0, _EH)], gsem[b])
        pltpu.async_copy(y_hbm.at[eidx.at[m, 0, pl.ds(_EH, _EH)]],
                         erows.at[b, pl.ds(_EH, _EH)], gsem[b])

    def g_wait(i, m, b):
        pltpu.make_async_copy(y_hbm.at[eidx.at[m, 0, pl.ds(0, _EH)]],
                              erows.at[b, pl.ds(0, _EH)], gsem[b]).wait()
        pltpu.make_async_copy(y_hbm.at[eidx.at[m, 0, pl.ds(_EH, _EH)]],
                              erows.at[b, pl.ds(_EH, _EH)], gsem[b]).wait()

    def s_start(i, m, b):
        pltpu.async_copy(erows.at[b], acc.at[eidx.at[m, 1]], ssem[b],
                         add=True)

    def s_wait(i, m, b):
        pltpu.make_async_copy(erows.at[b], acc.at[eidx.at[m, 1]],
                              ssem[b]).wait()


    for j in range(4):
        i_start(j, j)
    i_wait(0, 0)
    g_start(0, 0, 0)
    i_wait(1, 1)
    g_start(1, 1, 1)
    for i in range(6):
        g_wait(i, i % _NIB, i % _NBUF)
        s_start(i, i % _NIB, i % _NBUF)
        if i >= 1:
            s_wait(i - 1, (i - 1) % _NIB, (i - 1) % _NBUF)
        i_wait(i + 2, (i + 2) % _NIB)
        g_start(i + 2, (i + 2) % _NIB, (i + 2) % _NBUF)
        i_start(i + 4, (i + 4) % _NIB)

    def group(G, carry):
        for v in range(6):
            i = G * 6 + v
            g_wait(i, v, v % _NBUF)
            s_start(i, v, v % _NBUF)
            s_wait(i - 1, (v - 1) % _NIB, (v - 1) % _NBUF)
            i_wait(i + 2, (v + 2) % _NIB)
            g_start(i + 2, (v + 2) % _NIB, (v + 2) % _NBUF)
            i_start(i + 4, (v + 4) % _NIB)
        return carry

    lax.fori_loop(1, 16, group, 0)
    for i in range(96, _ESTEPS):
        g_wait(i, i % _NIB, i % _NBUF)
        s_start(i, i % _NIB, i % _NBUF)
        s_wait(i - 1, (i - 1) % _NIB, (i - 1) % _NBUF)
        if i + 2 < _ESTEPS:
            i_wait(i + 2, (i + 2) % _NIB)
            g_start(i + 2, (i + 2) % _NIB, (i + 2) % _NBUF)
    s_wait(_ESTEPS - 1, (_ESTEPS - 1) % _NIB, (_ESTEPS - 1) % _NBUF)
    plsc.subcore_barrier()
    pltpu.sync_copy(acc.at[pl.ds(r0, _RPT)],
                    out_hbm.at[cid, pl.ds(r0, _RPT)])


def _xw_body(hp_ref, x_ref, w_ref, y_ref, dinv_ref):
    deg = hp_ref[0, :] + hp_ref[1, :] + 1.0
    dinv = lax.rsqrt(deg)
    dinv_ref[0, 0, :] = dinv
    xw = jnp.dot(x_ref[...], w_ref[...], preferred_element_type=jnp.float32)
    y_ref[...] = xw * dinv[:, None]


def _layer_body(p_ref, y_ref, dinv_ref, b_ref, w_ref, ynext_ref):
    comb = p_ref[0] + p_ref[1] - y_ref[...]
    dinv = dinv_ref[0, 0, :]
    h = jnp.maximum(comb * dinv[:, None] + b_ref[0], 0.0)
    hw = jnp.dot(h, w_ref[...], preferred_element_type=jnp.float32)
    ynext_ref[...] = hw * dinv[:, None]


def _head_body(p_ref, y_ref, dinv_ref, b_ref, batch_ref, fw0_ref, fb0_ref,
               fw1_ref, fb1_ref, wout_ref, bout_ref, out_ref, seg_scr,
               cnt_scr):
    i = pl.program_id(0)

    @pl.when(i == 0)
    def _init():
        seg_scr[...] = jnp.zeros_like(seg_scr)
        cnt_scr[...] = jnp.zeros_like(cnt_scr)

    comb = p_ref[0] + p_ref[1] - y_ref[...]
    dinv = dinv_ref[0, 0, :]
    h = jnp.maximum(comb * dinv[:, None] + b_ref[0], 0.0)
    bvec = batch_ref[0, 0, :]
    bids = lax.broadcasted_iota(jnp.int32, (_B, _RB), 0)
    onehot = (bvec[None, :] == bids).astype(jnp.float32)
    seg_scr[...] += jnp.dot(onehot, h, preferred_element_type=jnp.float32)
    cnt_scr[...] += jnp.broadcast_to(
        jnp.sum(onehot, axis=1)[:, None], (_B, _H))

    @pl.when(i == _GB - 1)
    def _fin():
        g = seg_scr[...] / jnp.maximum(cnt_scr[...], 1.0)
        g = jnp.maximum(
            jnp.dot(g, fw0_ref[...], preferred_element_type=jnp.float32) + fb0_ref[0], 0.0)
        g = jnp.maximum(
            jnp.dot(g, fw1_ref[...], preferred_element_type=jnp.float32) + fb1_ref[0], 0.0)
        out_ref[...] = jnp.dot(
            g, wout_ref[...],
            preferred_element_type=jnp.float32) + bout_ref[0]


_xw_call = pl.pallas_call(
    _xw_body,
    grid=(_GB,),
    in_specs=[
        pl.BlockSpec((_NC, _RB), lambda i: (0, i)),
        pl.BlockSpec((_RB, _H), lambda i: (i, 0)),
        pl.BlockSpec((_H, _H), lambda i: (0, 0)),
    ],
    out_specs=[
        pl.BlockSpec((_RB, _H), lambda i: (i, 0)),
        pl.BlockSpec((1, 1, _RB), lambda i: (i, 0, 0)),
    ],
    out_shape=[
        jax.ShapeDtypeStruct((_NP, _H), jnp.float32),
        jax.ShapeDtypeStruct((_GB, 1, _RB), jnp.float32),
    ],
)

_layer_call = pl.pallas_call(
    _layer_body,
    grid=(_GB,),
    in_specs=[
        pl.BlockSpec((_NC, _RB, _H), lambda i: (0, i, 0)),
        pl.BlockSpec((_RB, _H), lambda i: (i, 0)),
        pl.BlockSpec((1, 1, _RB), lambda i: (i, 0, 0)),
        pl.BlockSpec((1, _H), lambda i: (0, 0)),
        pl.BlockSpec((_H, _H), lambda i: (0, 0)),
    ],
    out_specs=pl.BlockSpec((_RB, _H), lambda i: (i, 0)),
    out_shape=jax.ShapeDtypeStruct((_NP, _H), jnp.float32),
)

_head_call = pl.pallas_call(
    _head_body,
    grid=(_GB,),
    in_specs=[
        pl.BlockSpec((_NC, _RB, _H), lambda i: (0, i, 0)),
        pl.BlockSpec((_RB, _H), lambda i: (i, 0)),
        pl.BlockSpec((1, 1, _RB), lambda i: (i, 0, 0)),
        pl.BlockSpec((1, _H), lambda i: (0, 0)),
        pl.BlockSpec((1, 1, _RB), lambda i: (i, 0, 0)),
        pl.BlockSpec((_H, _H), lambda i: (0, 0)),
        pl.BlockSpec((1, _H), lambda i: (0, 0)),
        pl.BlockSpec((_H, _H), lambda i: (0, 0)),
        pl.BlockSpec((1, _H), lambda i: (0, 0)),
        pl.BlockSpec((_H, 2), lambda i: (0, 0)),
        pl.BlockSpec((1, 2), lambda i: (0, 0)),
    ],
    out_specs=pl.BlockSpec((_B, 2), lambda i: (0, 0)),
    out_shape=jax.ShapeDtypeStruct((_B, 2), jnp.float32),
    scratch_shapes=[
        pltpu.VMEM((_B, _H), jnp.float32),
        pltpu.VMEM((_B, _H), jnp.float32),
    ],
)


def kernel(x, edge_index, batch, W0, b0, W1, b1, Fw0, Fb0, Fw1, Fb1, Wout,
           bout):
    e3 = edge_index.reshape(2, _NW, _ESTEPS, _EK)
    xp = jnp.pad(x, ((0, _NP - _N), (0, 0)))
    hp = _deg_kernel(e3)
    y0, dinv = _xw_call(hp, xp, W0)
    p0 = _agg_kernel(y0, e3)
    y1 = _layer_call(p0, y0, dinv, b0.reshape(1, _H), W1)
    p1 = _agg_kernel(y1, e3)
    batch_r = jnp.pad(batch, (0, _NP - _N),
                      constant_values=_B).reshape(_GB, 1, _RB)
    return _head_call(p1, y1, dinv, b1.reshape(1, _H), batch_r, Fw0,
                      Fb0.reshape(1, _H), Fw1, Fb1.reshape(1, _H), Wout,
                      bout.reshape(1, 2))


# speedup vs baseline: 1.0089x; 1.0089x over previous
import functools

import jax
import jax.numpy as jnp
from jax import lax
from jax.experimental import pallas as pl
from jax.experimental.pallas import tpu as pltpu
from jax.experimental.pallas import tpu_sc as plsc

_N = 10000
_NP = 10240
_E = 320000
_H = 128
_B = 64

_NC = 2
_NS = 16
_NW = _NC * _NS
_EPT = _E // _NW
_EK = 100
_ESTEPS = _EPT // _EK
_NBUF = 3
_NIB = 6
_RPT = _NP // _NS
_RK = 128
_RSTEPS = _RPT // _RK
_HPT = _NP // _NS

_RB = 1024
_GB = _NP // _RB

_mesh = plsc.VectorSubcoreMesh(core_axis_name="c", subcore_axis_name="s")


@functools.partial(
    pl.kernel,
    mesh=_mesh,
    out_type=jax.ShapeDtypeStruct((_NC, _NP), jnp.float32),
    scratch_types=[
        pltpu.VMEM((_HPT,), jnp.float32),
        pltpu.VMEM((112,), jnp.float32),
        pltpu.VMEM((_ESTEPS, _EK), jnp.int32),
        pltpu.SemaphoreType.DMA,
        pltpu.VMEM_SHARED((_NP,), jnp.float32),
    ],
)
def _deg_kernel(e3_hbm, out_hbm, zbuf, ones_v, dstall, dsem, hist):
    cid = lax.axis_index("c")
    sid = lax.axis_index("s")
    wid = cid * _NS + sid
    pltpu.sync_copy(e3_hbm.at[1, wid], dstall)
    for j in range(_HPT // 16):
        zbuf[pl.ds(j * 16, 16)] = jnp.zeros((16,), jnp.float32)
    for j in range(112 // 16):
        ones_v[pl.ds(j * 16, 16)] = jnp.ones((16,), jnp.float32)
    pltpu.sync_copy(zbuf, hist.at[pl.ds(sid * _HPT, _HPT)])
    plsc.subcore_barrier()
    ones_c = ones_v.at[pl.ds(0, _EK)]

    def fire(i, carry):
        pltpu.async_copy(ones_c, hist.at[dstall.at[i]], dsem, add=True)
        return carry

    def drain(i, carry):
        pltpu.make_async_copy(ones_c, hist.at[dstall.at[i]], dsem).wait()
        return carry

    lax.fori_loop(0, _ESTEPS, fire, 0)
    lax.fori_loop(0, _ESTEPS, drain, 0)
    plsc.subcore_barrier()
    pltpu.sync_copy(hist.at[pl.ds(sid * _HPT, _HPT)], zbuf)
    pltpu.sync_copy(zbuf, out_hbm.at[cid, pl.ds(sid * _HPT, _HPT)])


@functools.partial(
    pl.kernel,
    mesh=_mesh,
    out_type=jax.ShapeDtypeStruct((_NC, _NP, _H), jnp.float32),
    scratch_types=[
        pltpu.VMEM((_NBUF, _EK, _H), jnp.float32),
        pltpu.VMEM((_NIB, 2, _EK), jnp.int32),
    ] + [pltpu.SemaphoreType.DMA] * (2 * _NBUF + _NIB) + [
        pltpu.VMEM_SHARED((_NP, _H), jnp.float32),
    ],
)
def _agg_kernel(y_hbm, e3_hbm, out_hbm, erows, eidx,
                g0, g1, g2, s0, s1, s2,
                i0, i1, i2, i3, i4, i5, acc):
    gsem = [g0, g1, g2]
    ssem = [s0, s1, s2]
    isem = [i0, i1, i2, i3, i4, i5]
    cid = lax.axis_index("c")
    sid = lax.axis_index("s")
    wid = cid * _NS + sid
    r0 = sid * _RPT
    pltpu.sync_copy(y_hbm.at[pl.ds(r0, _RPT)], acc.at[pl.ds(r0, _RPT)])
    plsc.subcore_barrier()

    def i_start(j, m):
        pltpu.async_copy(e3_hbm.at[0, wid, j], eidx.at[m, 0], isem[m])
        pltpu.async_copy(e3_hbm.at[1, wid, j], eidx.at[m, 1], isem[m])

    def i_wait(j, m):
        pltpu.make_async_copy(e3_hbm.at[0, wid, j], eidx.at[m, 0],
                              isem[m]).wait()
        pltpu.make_async_copy(e3_hbm.at[1, wid, j], eidx.at[m, 1],
                              isem[m]).wait()

    def g_start(i, m, b):
        pltpu.async_copy(y_hbm.at[eidx.at[m, 0]], erows.at[b], gsem[b])

    def g_wait(i, m, b):
        pltpu.make_async_copy(y_hbm.at[eidx.at[m, 0]], erows.at[b],
                              gsem[b]).wait()

    def s_start(i, m, b):
        pltpu.async_copy(erows.at[b], acc.at[eidx.at[m, 1]], ssem[b],
                         add=True)

    def s_wait(i, m, b):
        pltpu.make_async_copy(erows.at[b], acc.at[eidx.at[m, 1]],
                              ssem[b]).wait()


    for j in range(4):
        i_start(j, j)
    i_wait(0, 0)
    g_start(0, 0, 0)
    i_wait(1, 1)
    g_start(1, 1, 1)
    for i in range(6):
        g_wait(i, i % _NIB, i % _NBUF)
        s_start(i, i % _NIB, i % _NBUF)
        if i >= 1:
            s_wait(i - 1, (i - 1) % _NIB, (i - 1) % _NBUF)
        i_wait(i + 2, (i + 2) % _NIB)
        g_start(i + 2, (i + 2) % _NIB, (i + 2) % _NBUF)
        i_start(i + 4, (i + 4) % _NIB)

    def group(G, carry):
        for v in range(6):
            i = G * 6 + v
            g_wait(i, v, v % _NBUF)
            s_start(i, v, v % _NBUF)
            s_wait(i - 1, (v - 1) % _NIB, (v - 1) % _NBUF)
            i_wait(i + 2, (v + 2) % _NIB)
            g_start(i + 2, (v + 2) % _NIB, (v + 2) % _NBUF)
            i_start(i + 4, (v + 4) % _NIB)
        return carry

    lax.fori_loop(1, 16, group, 0)
    for i in range(96, _ESTEPS):
        g_wait(i, i % _NIB, i % _NBUF)
        s_start(i, i % _NIB, i % _NBUF)
        s_wait(i - 1, (i - 1) % _NIB, (i - 1) % _NBUF)
        if i + 2 < _ESTEPS:
            i_wait(i + 2, (i + 2) % _NIB)
            g_start(i + 2, (i + 2) % _NIB, (i + 2) % _NBUF)
    s_wait(_ESTEPS - 1, (_ESTEPS - 1) % _NIB, (_ESTEPS - 1) % _NBUF)
    plsc.subcore_barrier()
    pltpu.sync_copy(acc.at[pl.ds(r0, _RPT)],
                    out_hbm.at[cid, pl.ds(r0, _RPT)])


def _xw_body(hp_ref, x_ref, w_ref, y_ref, dinv_ref):
    deg = hp_ref[0, :] + hp_ref[1, :] + 1.0
    dinv = lax.rsqrt(deg)
    dinv_ref[0, 0, :] = dinv
    xw = jnp.dot(x_ref[...], w_ref[...], preferred_element_type=jnp.float32)
    y_ref[...] = xw * dinv[:, None]


def _layer_body(p_ref, y_ref, dinv_ref, b_ref, w_ref, ynext_ref):
    comb = p_ref[0] + p_ref[1] - y_ref[...]
    dinv = dinv_ref[0, 0, :]
    h = jnp.maximum(comb * dinv[:, None] + b_ref[0], 0.0)
    hw = jnp.dot(h, w_ref[...], preferred_element_type=jnp.float32)
    ynext_ref[...] = hw * dinv[:, None]


def _head_body(p_ref, y_ref, dinv_ref, b_ref, batch_ref, fw0_ref, fb0_ref,
               fw1_ref, fb1_ref, wout_ref, bout_ref, out_ref, seg_scr,
               cnt_scr):
    i = pl.program_id(0)

    @pl.when(i == 0)
    def _init():
        seg_scr[...] = jnp.zeros_like(seg_scr)
        cnt_scr[...] = jnp.zeros_like(cnt_scr)

    comb = p_ref[0] + p_ref[1] - y_ref[...]
    dinv = dinv_ref[0, 0, :]
    h = jnp.maximum(comb * dinv[:, None] + b_ref[0], 0.0)
    bvec = batch_ref[0, 0, :]
    bids = lax.broadcasted_iota(jnp.int32, (_B, _RB), 0)
    onehot = (bvec[None, :] == bids).astype(jnp.float32)
    seg_scr[...] += jnp.dot(onehot, h, preferred_element_type=jnp.float32)
    cnt_scr[...] += jnp.broadcast_to(
        jnp.sum(onehot, axis=1)[:, None], (_B, _H))

    @pl.when(i == _GB - 1)
    def _fin():
        g = seg_scr[...] / jnp.maximum(cnt_scr[...], 1.0)
        g = jnp.maximum(
            jnp.dot(g, fw0_ref[...], preferred_element_type=jnp.float32) + fb0_ref[0], 0.0)
        g = jnp.maximum(
            jnp.dot(g, fw1_ref[...], preferred_element_type=jnp.float32) + fb1_ref[0], 0.0)
        out_ref[...] = jnp.dot(
            g, wout_ref[...],
            preferred_element_type=jnp.float32) + bout_ref[0]


_xw_call = pl.pallas_call(
    _xw_body,
    grid=(_GB,),
    in_specs=[
        pl.BlockSpec((_NC, _RB), lambda i: (0, i)),
        pl.BlockSpec((_RB, _H), lambda i: (i, 0)),
        pl.BlockSpec((_H, _H), lambda i: (0, 0)),
    ],
    out_specs=[
        pl.BlockSpec((_RB, _H), lambda i: (i, 0)),
        pl.BlockSpec((1, 1, _RB), lambda i: (i, 0, 0)),
    ],
    out_shape=[
        jax.ShapeDtypeStruct((_NP, _H), jnp.float32),
        jax.ShapeDtypeStruct((_GB, 1, _RB), jnp.float32),
    ],
)

_layer_call = pl.pallas_call(
    _layer_body,
    grid=(_GB,),
    in_specs=[
        pl.BlockSpec((_NC, _RB, _H), lambda i: (0, i, 0)),
        pl.BlockSpec((_RB, _H), lambda i: (i, 0)),
        pl.BlockSpec((1, 1, _RB), lambda i: (i, 0, 0)),
        pl.BlockSpec((1, _H), lambda i: (0, 0)),
        pl.BlockSpec((_H, _H), lambda i: (0, 0)),
    ],
    out_specs=pl.BlockSpec((_RB, _H), lambda i: (i, 0)),
    out_shape=jax.ShapeDtypeStruct((_NP, _H), jnp.float32),
)

_head_call = pl.pallas_call(
    _head_body,
    grid=(_GB,),
    in_specs=[
        pl.BlockSpec((_NC, _RB, _H), lambda i: (0, i, 0)),
        pl.BlockSpec((_RB, _H), lambda i: (i, 0)),
        pl.BlockSpec((1, 1, _RB), lambda i: (i, 0, 0)),
        pl.BlockSpec((1, _H), lambda i: (0, 0)),
        pl.BlockSpec((1, 1, _RB), lambda i: (i, 0, 0)),
        pl.BlockSpec((_H, _H), lambda i: (0, 0)),
        pl.BlockSpec((1, _H), lambda i: (0, 0)),
        pl.BlockSpec((_H, _H), lambda i: (0, 0)),
        pl.BlockSpec((1, _H), lambda i: (0, 0)),
        pl.BlockSpec((_H, 2), lambda i: (0, 0)),
        pl.BlockSpec((1, 2), lambda i: (0, 0)),
    ],
    out_specs=pl.BlockSpec((_B, 2), lambda i: (0, 0)),
    out_shape=jax.ShapeDtypeStruct((_B, 2), jnp.float32),
    scratch_shapes=[
        pltpu.VMEM((_B, _H), jnp.float32),
        pltpu.VMEM((_B, _H), jnp.float32),
    ],
)


def kernel(x, edge_index, batch, W0, b0, W1, b1, Fw0, Fb0, Fw1, Fb1, Wout,
           bout):
    e3 = edge_index.reshape(2, _NW, _ESTEPS, _EK)
    xp = jnp.pad(x, ((0, _NP - _N), (0, 0)))
    hp = _deg_kernel(e3)
    y0, dinv = _xw_call(hp, xp, W0)
    p0 = _agg_kernel(y0, e3)
    y1 = _layer_call(p0, y0, dinv, b0.reshape(1, _H), W1)
    p1 = _agg_kernel(y1, e3)
    batch_r = jnp.pad(batch, (0, _NP - _N),
                      constant_values=_B).reshape(_GB, 1, _RB)
    return _head_call(p1, y1, dinv, b1.reshape(1, _H), batch_r, Fw0,
                      Fb0.reshape(1, _H), Fw1, Fb1.reshape(1, _H), Wout,
                      bout.reshape(1, 2))
